# baseline (speedup 1.0000x reference)
import functools

import jax
import jax.numpy as jnp
from jax import lax
from jax.experimental import pallas as pl
from jax.experimental.pallas import tpu as pltpu
from jax.experimental.pallas import tpu_sc as plsc

N = 10000
D = 128
R = 8
E = 320000

NC = 2
NS = 16
NW = NC * NS
E_PER_W = E // NW
CHUNK = 80
NCHUNK = E_PER_W // CHUNK
NTRIPLE = 40
ZROWS = 16
S_STRIDE = 624
S_ROWS = 640

def _proj_body(h_ref, w_ref, out_ref):
    for rr in range(R):
        y = jnp.dot(h_ref[...], w_ref[rr],
                    preferred_element_type=jnp.float32)
        yi = pltpu.bitcast(y, jnp.int32)
        rbits = (yi + 0x7FFF + ((yi >> 16) & 1)) >> 16
        for c in range(D // 32):
            lo = rbits[:, c * 32:c * 32 + 16]
            hi = rbits[:, c * 32 + 16:c * 32 + 32]
            out_ref[rr, :, c * 16:(c + 1) * 16] = (lo & 0xFFFF) | (hi << 16)


def _project(h, W):
    BLK = 1000
    return pl.pallas_call(
        _proj_body,
        grid=(N // BLK,),
        in_specs=[
            pl.BlockSpec((BLK, D), lambda bi: (bi, 0)),
            pl.BlockSpec((R, D, D), lambda bi: (0, 0, 0)),
        ],
        out_specs=pl.BlockSpec((R, BLK, D // 2), lambda bi: (0, bi, 0)),
        out_shape=jax.ShapeDtypeStruct((R, N, D // 2), jnp.int32),
    )(h, W)


def _sc_edge_body(ei_hbm, r_hbm, norm_hbm, hall_hbm, out_hbm,
                  srcb, rb, idxc, dstb, nrm, rows_pk, rows_f32,
                  zero_v, agg_sh, gsem, ssem, xsem):
    cid = lax.axis_index("c")
    sid = lax.axis_index("s")
    wid = cid * NS + sid
    base = wid * E_PER_W

    def zero_body(i, carry):
        for c in range(D // 16):
            zero_v[i, pl.ds(c * 16, 16)] = jnp.zeros((16,), jnp.float32)
        return carry
    lax.fori_loop(0, ZROWS, zero_body, 0)

    def zcopy_body(j, carry):
        pltpu.sync_copy(
            zero_v, agg_sh.at[pl.ds(sid * S_STRIDE + j * ZROWS, ZROWS)])
        return carry
    lax.fori_loop(0, S_ROWS // ZROWS, zcopy_body, 0)
    plsc.subcore_barrier()

    def stage_off(t):
        return jnp.minimum(t, NCHUNK - 1) * CHUNK

    def issue_stage(t, b):
        off = stage_off(t)
        pltpu.async_copy(
            ei_hbm.at[pl.ds(base + off, CHUNK)], srcb[b], xsem[b])
        pltpu.async_copy(
            r_hbm.at[pl.ds(base + off, CHUNK)], rb[b], xsem[b])

    def wait_stage(t, b):
        off = stage_off(t)
        pltpu.make_async_copy(
            ei_hbm.at[pl.ds(base + off, CHUNK)], srcb[b], xsem[b]).wait()
        pltpu.make_async_copy(
            r_hbm.at[pl.ds(base + off, CHUNK)], rb[b], xsem[b]).wait()

    def compute_idx(b):
        for g in range(CHUNK // 16):
            s = srcb[b][pl.ds(g * 16, 16)]
            rr = rb[b][pl.ds(g * 16, 16)]
            idxc[b][pl.ds(g * 16, 16)] = rr * N + s

    def issue_g(t, b):
        off = t * CHUNK
        pltpu.async_copy(hall_hbm.at[idxc[b]], rows_pk[b], gsem[b])
        pltpu.async_copy(
            norm_hbm.at[pl.ds(base + off, CHUNK)], nrm[b], gsem[b])
        pltpu.async_copy(
            ei_hbm.at[pl.ds(E + base + off, CHUNK)], dstb[b], gsem[b])

    def wait_g(t, b):
        off = t * CHUNK
        pltpu.make_async_copy(hall_hbm.at[idxc[b]], rows_pk[b],
                              gsem[b]).wait()
        pltpu.make_async_copy(
            norm_hbm.at[pl.ds(base + off, CHUNK)], nrm[b], gsem[b]).wait()
        pltpu.make_async_copy(
            ei_hbm.at[pl.ds(E + base + off, CHUNK)], dstb[b],
            gsem[b]).wait()

    hi_mask = jnp.full((16,), -65536, dtype=jnp.int32)

    def scale(b):
        def group_body(g, c2):
            nv = nrm[b][pl.ds(g * 16, 16)]
            for k in range(16):
                nb = nv[k]
                e = g * 16 + k
                for c in range(D // 32):
                    v = rows_pk[b][e, pl.ds(c * 16, 16)]
                    lo = plsc.bitcast(v << 16, jnp.float32)
                    hi = plsc.bitcast(v & hi_mask, jnp.float32)
                    rows_f32[b][e, pl.ds(c * 32, 16)] = lo * nb
                    rows_f32[b][e, pl.ds(c * 32 + 16, 16)] = hi * nb
            return c2
        lax.fori_loop(0, CHUNK // 16, group_body, 0)

    def issue_s(b):
        pltpu.async_copy(rows_f32[b], agg_sh.at[dstb[b]], ssem[b], add=True)

    def wait_s(b):
        pltpu.make_async_copy(rows_f32[b], agg_sh.at[dstb[b]],
                              ssem[b]).wait()

    for b in range(3):
        issue_stage(b, b)
    for b in range(3):
        wait_stage(b, b)
        compute_idx(b)
        issue_stage(b + 3, b)
        issue_g(b, b)
    wait_g(0, 0); scale(0); issue_s(0)
    wait_stage(3, 0); compute_idx(0); issue_stage(6, 0)

    def station(u, b, b2, last_g, do_stage):
        wait_g(u, b); scale(b); issue_s(b)
        wait_stage(u + 3, b)
        compute_idx(b)
        if do_stage:
            issue_stage(u + 6, b)
        if last_g is not None:
            wait_s(b2)
            issue_g(last_g, b2)

    def triple_body(i, carry):
        u = 3 * i + 1
        station(u, 1, 0, u + 2, True)
        station(u + 1, 2, 1, u + 3, True)
        station(u + 2, 0, 2, u + 4, True)
        return carry
    lax.fori_loop(0, NTRIPLE, triple_body, 0)

    wait_g(121, 1); scale(1); issue_s(1)
    wait_stage(124, 1); compute_idx(1)
    wait_s(0); issue_g(123, 0)
    wait_g(122, 2); scale(2); issue_s(2)
    wait_stage(125, 2)
    wait_s(1); issue_g(124, 1)
    wait_g(123, 0); scale(0); issue_s(0)
    wait_stage(126, 0)
    wait_s(2)
    wait_g(124, 1); scale(1); issue_s(1)
    wait_s(0)
    wait_s(1)

    plsc.subcore_barrier()
    pltpu.sync_copy(
        agg_sh.at[pl.ds(sid * S_STRIDE, S_ROWS)],
        out_hbm.at[cid, pl.ds(sid * S_STRIDE, S_ROWS)])


def _sc_edges(ei_flat, r, norm_flat, h_all):
    mesh = plsc.VectorSubcoreMesh(core_axis_name="c", subcore_axis_name="s")
    fn = functools.partial(
        pl.kernel, mesh=mesh,
        compiler_params=pltpu.CompilerParams(
            needs_layout_passes=False, use_tc_tiling_on_sc=False),
        out_type=jax.ShapeDtypeStruct((NC, N, D), jnp.float32),
        scratch_types=[
            [pltpu.VMEM((CHUNK,), jnp.int32)] * 3,
            [pltpu.VMEM((CHUNK,), jnp.int32)] * 3,
            [pltpu.VMEM((CHUNK,), jnp.int32)] * 3,
            [pltpu.VMEM((CHUNK,), jnp.int32)] * 3,
            [pltpu.VMEM((CHUNK,), jnp.float32)] * 3,
            [pltpu.VMEM((CHUNK, D // 2), jnp.int32)] * 3,
            [pltpu.VMEM((CHUNK, D), jnp.float32)] * 3,
            pltpu.VMEM((ZROWS, D), jnp.float32),
            pltpu.VMEM_SHARED((N, D), jnp.float32),
            [pltpu.SemaphoreType.DMA] * 3,
            [pltpu.SemaphoreType.DMA] * 3,
            [pltpu.SemaphoreType.DMA] * 3,
        ],
    )(_sc_edge_body)
    return fn(ei_flat, r, norm_flat, h_all)


def _combine_body(p_ref, out_ref):
    out_ref[...] = jnp.maximum(p_ref[0] + p_ref[1], 0.0)


def _combine(partials):
    BLK = 400
    return pl.pallas_call(
        _combine_body,
        grid=(N // BLK,),
        in_specs=[pl.BlockSpec((NC, BLK, D), lambda bi: (0, bi, 0))],
        out_specs=pl.BlockSpec((BLK, D), lambda bi: (bi, 0)),
        out_shape=jax.ShapeDtypeStruct((N, D), jnp.float32),
    )(partials)


def kernel(edge_index, h, r, norm, W):
    norm_flat = norm.reshape(E)
    h_all = _project(h, W).reshape(R * N, D // 2)
    partials = _sc_edges(edge_index.reshape(2 * E), r, norm_flat, h_all)
    return _combine(partials)


# speedup vs baseline: 1.9606x; 1.9606x over previous
import functools

import jax
import jax.numpy as jnp
from jax import lax
from jax.experimental import pallas as pl
from jax.experimental.pallas import tpu as pltpu
from jax.experimental.pallas import tpu_sc as plsc

N = 10000
D = 128
R = 8
E = 320000

NC = 2
NS = 16
NW = NC * NS
E_PER_W = E // NW
CHUNK = 80
NCHUNK = E_PER_W // CHUNK
NTRIPLE = 40
ZROWS = 16
S_STRIDE = 624
S_ROWS = 640

def _proj_body(h_ref, w_ref, out_ref):
    for rr in range(R):
        out_ref[rr] = jnp.dot(h_ref[...], w_ref[rr],
                              preferred_element_type=jnp.float32)


def _project(h, W):
    BLK = 1000
    return pl.pallas_call(
        _proj_body,
        grid=(N // BLK,),
        in_specs=[
            pl.BlockSpec((BLK, D), lambda bi: (bi, 0)),
            pl.BlockSpec((R, D, D), lambda bi: (0, 0, 0)),
        ],
        out_specs=pl.BlockSpec((R, BLK, D), lambda bi: (0, bi, 0)),
        out_shape=jax.ShapeDtypeStruct((R, N, D), jnp.float32),
    )(h, W)


def _sc_edge_body(ei_hbm, r_hbm, norm_hbm, hall_hbm, out_hbm,
                  srcb, rb, idxc, dstb, nrm, rows,
                  zero_v, agg_sh, gsem, ssem, xsem):
    cid = lax.axis_index("c")
    sid = lax.axis_index("s")
    wid = cid * NS + sid
    base = wid * E_PER_W

    def zero_body(i, carry):
        for c in range(D // 16):
            zero_v[i, pl.ds(c * 16, 16)] = jnp.zeros((16,), jnp.float32)
        return carry
    lax.fori_loop(0, ZROWS, zero_body, 0)

    def zcopy_body(j, carry):
        pltpu.sync_copy(
            zero_v, agg_sh.at[pl.ds(sid * S_STRIDE + j * ZROWS, ZROWS)])
        return carry
    lax.fori_loop(0, S_ROWS // ZROWS, zcopy_body, 0)
    plsc.subcore_barrier()

    def stage_off(t):
        return jnp.minimum(t, NCHUNK - 1) * CHUNK

    def issue_stage(t, b):
        off = stage_off(t)
        pltpu.async_copy(
            ei_hbm.at[pl.ds(base + off, CHUNK)], srcb[b], xsem[b])
        pltpu.async_copy(
            r_hbm.at[pl.ds(base + off, CHUNK)], rb[b], xsem[b])

    def wait_stage(t, b):
        off = stage_off(t)
        pltpu.make_async_copy(
            ei_hbm.at[pl.ds(base + off, CHUNK)], srcb[b], xsem[b]).wait()
        pltpu.make_async_copy(
            r_hbm.at[pl.ds(base + off, CHUNK)], rb[b], xsem[b]).wait()

    def compute_idx(b):
        for g in range(CHUNK // 16):
            s = srcb[b][pl.ds(g * 16, 16)]
            rr = rb[b][pl.ds(g * 16, 16)]
            idxc[b][pl.ds(g * 16, 16)] = rr * N + s

    def issue_g(t, b):
        off = t * CHUNK
        pltpu.async_copy(hall_hbm.at[idxc[b]], rows[b], gsem[b])
        pltpu.async_copy(
            norm_hbm.at[pl.ds(base + off, CHUNK)], nrm[b], gsem[b])
        pltpu.async_copy(
            ei_hbm.at[pl.ds(E + base + off, CHUNK)], dstb[b], gsem[b])

    def wait_g(t, b):
        off = t * CHUNK
        pltpu.make_async_copy(hall_hbm.at[idxc[b]], rows[b],
                              gsem[b]).wait()
        pltpu.make_async_copy(
            norm_hbm.at[pl.ds(base + off, CHUNK)], nrm[b], gsem[b]).wait()
        pltpu.make_async_copy(
            ei_hbm.at[pl.ds(E + base + off, CHUNK)], dstb[b],
            gsem[b]).wait()

    def scale(b):
        def group_body(g, c2):
            nv = nrm[b][pl.ds(g * 16, 16)]
            for k in range(16):
                nb = nv[k]
                e = g * 16 + k
                for c in range(D // 16):
                    rows[b][e, pl.ds(c * 16, 16)] = (
                        rows[b][e, pl.ds(c * 16, 16)] * nb)
            return c2
        lax.fori_loop(0, CHUNK // 16, group_body, 0)

    def issue_s(b):
        pltpu.async_copy(rows[b], agg_sh.at[dstb[b]], ssem[b], add=True)

    def wait_s(b):
        pltpu.make_async_copy(rows[b], agg_sh.at[dstb[b]],
                              ssem[b]).wait()

    for b in range(3):
        issue_stage(b, b)
    for b in range(3):
        wait_stage(b, b)
        compute_idx(b)
        issue_stage(b + 3, b)
        issue_g(b, b)
    wait_g(0, 0); scale(0); issue_s(0)
    wait_stage(3, 0); compute_idx(0); issue_stage(6, 0)

    def station(u, b, b2, last_g, do_stage):
        wait_g(u, b); scale(b); issue_s(b)
        wait_stage(u + 3, b)
        compute_idx(b)
        if do_stage:
            issue_stage(u + 6, b)
        if last_g is not None:
            wait_s(b2)
            issue_g(last_g, b2)

    def triple_body(i, carry):
        u = 3 * i + 1
        station(u, 1, 0, u + 2, True)
        station(u + 1, 2, 1, u + 3, True)
        station(u + 2, 0, 2, u + 4, True)
        return carry
    lax.fori_loop(0, NTRIPLE, triple_body, 0)

    wait_g(121, 1); scale(1); issue_s(1)
    wait_stage(124, 1); compute_idx(1)
    wait_s(0); issue_g(123, 0)
    wait_g(122, 2); scale(2); issue_s(2)
    wait_stage(125, 2)
    wait_s(1); issue_g(124, 1)
    wait_g(123, 0); scale(0); issue_s(0)
    wait_stage(126, 0)
    wait_s(2)
    wait_g(124, 1); scale(1); issue_s(1)
    wait_s(0)
    wait_s(1)

    plsc.subcore_barrier()
    pltpu.sync_copy(
        agg_sh.at[pl.ds(sid * S_STRIDE, S_ROWS)],
        out_hbm.at[cid, pl.ds(sid * S_STRIDE, S_ROWS)])


def _sc_edges(ei_flat, r, norm_flat, h_all):
    mesh = plsc.VectorSubcoreMesh(core_axis_name="c", subcore_axis_name="s")
    fn = functools.partial(
        pl.kernel, mesh=mesh,
        out_type=jax.ShapeDtypeStruct((NC, N, D), jnp.float32),
        scratch_types=[
            [pltpu.VMEM((CHUNK,), jnp.int32)] * 3,
            [pltpu.VMEM((CHUNK,), jnp.int32)] * 3,
            [pltpu.VMEM((CHUNK,), jnp.int32)] * 3,
            [pltpu.VMEM((CHUNK,), jnp.int32)] * 3,
            [pltpu.VMEM((CHUNK,), jnp.float32)] * 3,
            [pltpu.VMEM((CHUNK, D), jnp.float32)] * 3,
            pltpu.VMEM((ZROWS, D), jnp.float32),
            pltpu.VMEM_SHARED((N, D), jnp.float32),
            [pltpu.SemaphoreType.DMA] * 3,
            [pltpu.SemaphoreType.DMA] * 3,
            [pltpu.SemaphoreType.DMA] * 3,
        ],
    )(_sc_edge_body)
    return fn(ei_flat, r, norm_flat, h_all)


def _combine_body(p_ref, out_ref):
    out_ref[...] = jnp.maximum(p_ref[0] + p_ref[1], 0.0)


def _combine(partials):
    BLK = 400
    return pl.pallas_call(
        _combine_body,
        grid=(N // BLK,),
        in_specs=[pl.BlockSpec((NC, BLK, D), lambda bi: (0, bi, 0))],
        out_specs=pl.BlockSpec((BLK, D), lambda bi: (bi, 0)),
        out_shape=jax.ShapeDtypeStruct((N, D), jnp.float32),
    )(partials)


def kernel(edge_index, h, r, norm, W):
    norm_flat = norm.reshape(E)
    h_all = _project(h, W).reshape(R * N, D)
    partials = _sc_edges(edge_index.reshape(2 * E), r, norm_flat, h_all)
    return _combine(partials)


# speedup vs baseline: 2.0514x; 1.0463x over previous
import functools

import jax
import jax.numpy as jnp
from jax import lax
from jax.experimental import pallas as pl
from jax.experimental.pallas import tpu as pltpu
from jax.experimental.pallas import tpu_sc as plsc

N = 10000
D = 128
R = 8
E = 320000

NC = 2
NS = 16
NW = NC * NS
E_PER_W = E // NW
CHUNK = 80
NCHUNK = E_PER_W // CHUNK
NTRIPLE = 40
ZROWS = 16
S_STRIDE = 624
S_ROWS = 640

def _proj_body(h_ref, w_ref, out_ref):
    for rr in range(R):
        out_ref[rr] = jnp.dot(h_ref[...], w_ref[rr],
                              preferred_element_type=jnp.float32)


def _project(h, W):
    BLK = 1000
    return pl.pallas_call(
        _proj_body,
        grid=(N // BLK,),
        in_specs=[
            pl.BlockSpec((BLK, D), lambda bi: (bi, 0)),
            pl.BlockSpec((R, D, D), lambda bi: (0, 0, 0)),
        ],
        out_specs=pl.BlockSpec((R, BLK, D), lambda bi: (0, bi, 0)),
        out_shape=jax.ShapeDtypeStruct((R, N, D), jnp.float32),
    )(h, W)


def _sc_edge_body(ei_hbm, r_hbm, norm_hbm, hall_hbm, out_hbm,
                  srcb, rb, idxc, dstb, nrm, rows,
                  zero_v, agg_sh, gsem, ssem, xsem):
    cid = lax.axis_index("c")
    sid = lax.axis_index("s")
    wid = cid * NS + sid
    base = wid * E_PER_W

    def zero_body(i, carry):
        for c in range(D // 16):
            zero_v[i, pl.ds(c * 16, 16)] = jnp.zeros((16,), jnp.float32)
        return carry
    lax.fori_loop(0, ZROWS, zero_body, 0)

    def zcopy_body(j, carry):
        pltpu.sync_copy(
            zero_v, agg_sh.at[pl.ds(sid * S_STRIDE + j * ZROWS, ZROWS)])
        return carry
    lax.fori_loop(0, S_ROWS // ZROWS, zcopy_body, 0)
    plsc.subcore_barrier()

    def stage_off(t):
        return jnp.minimum(t, NCHUNK - 1) * CHUNK

    def issue_stage(t, b):
        off = stage_off(t)
        pltpu.async_copy(
            ei_hbm.at[pl.ds(base + off, CHUNK)], srcb[b], xsem[b])
        pltpu.async_copy(
            r_hbm.at[pl.ds(base + off, CHUNK)], rb[b], xsem[b])

    def wait_stage(t, b):
        off = stage_off(t)
        pltpu.make_async_copy(
            ei_hbm.at[pl.ds(base + off, CHUNK)], srcb[b], xsem[b]).wait()
        pltpu.make_async_copy(
            r_hbm.at[pl.ds(base + off, CHUNK)], rb[b], xsem[b]).wait()

    def compute_idx(b):
        for g in range(CHUNK // 16):
            s = srcb[b][pl.ds(g * 16, 16)]
            rr = rb[b][pl.ds(g * 16, 16)]
            idxc[b][pl.ds(g * 16, 16)] = rr * N + s

    def issue_g(t, b):
        off = t * CHUNK
        pltpu.async_copy(hall_hbm.at[idxc[b]], rows[b], gsem[b])
        pltpu.async_copy(
            norm_hbm.at[pl.ds(base + off, CHUNK)], nrm[b], gsem[b])
        pltpu.async_copy(
            ei_hbm.at[pl.ds(E + base + off, CHUNK)], dstb[b], gsem[b])

    def wait_g(t, b):
        off = t * CHUNK
        pltpu.make_async_copy(hall_hbm.at[idxc[b]], rows[b],
                              gsem[b]).wait()
        pltpu.make_async_copy(
            norm_hbm.at[pl.ds(base + off, CHUNK)], nrm[b], gsem[b]).wait()
        pltpu.make_async_copy(
            ei_hbm.at[pl.ds(E + base + off, CHUNK)], dstb[b],
            gsem[b]).wait()

    def scale(b):
        def group_body(g, c2):
            nv = nrm[b][pl.ds(g * 16, 16)]
            for k in range(16):
                nb = nv[k]
                e = g * 16 + k
                for c in range(D // 16):
                    rows[b][e, pl.ds(c * 16, 16)] = (
                        rows[b][e, pl.ds(c * 16, 16)] * nb)
            return c2
        lax.fori_loop(0, CHUNK // 16, group_body, 0)

    def issue_s(b):
        pltpu.async_copy(rows[b], agg_sh.at[dstb[b]], ssem[b], add=True)

    def wait_s(b):
        pltpu.make_async_copy(rows[b], agg_sh.at[dstb[b]],
                              ssem[b]).wait()

    for b in range(3):
        issue_stage(b, b)
    for b in range(3):
        wait_stage(b, b)
        compute_idx(b)
        issue_stage(b + 3, b)
        issue_g(b, b)
    wait_g(0, 0); scale(0); issue_s(0)
    wait_stage(3, 0); compute_idx(0); issue_stage(6, 0)

    def station(u, b, b2, last_g, do_stage):
        wait_g(u, b); scale(b); issue_s(b)
        wait_stage(u + 3, b)
        compute_idx(b)
        if do_stage:
            issue_stage(u + 6, b)
        if last_g is not None:
            wait_s(b2)
            issue_g(last_g, b2)

    def triple_body(i, carry):
        u = 3 * i + 1
        station(u, 1, 0, u + 2, True)
        station(u + 1, 2, 1, u + 3, True)
        station(u + 2, 0, 2, u + 4, True)
        return carry
    lax.fori_loop(0, NTRIPLE, triple_body, 0)

    wait_g(121, 1); scale(1); issue_s(1)
    wait_stage(124, 1); compute_idx(1)
    wait_s(0); issue_g(123, 0)
    wait_g(122, 2); scale(2); issue_s(2)
    wait_stage(125, 2)
    wait_s(1); issue_g(124, 1)
    wait_g(123, 0); scale(0); issue_s(0)
    wait_stage(126, 0)
    wait_s(2)
    wait_g(124, 1); scale(1); issue_s(1)
    wait_s(0)
    wait_s(1)

    plsc.subcore_barrier()
    pltpu.sync_copy(
        agg_sh.at[pl.ds(sid * S_STRIDE, S_ROWS)],
        out_hbm.at[cid, pl.ds(sid * S_STRIDE, S_ROWS)])


def _sc_edges(ei_flat, r, norm_flat, h_all):
    mesh = plsc.VectorSubcoreMesh(core_axis_name="c", subcore_axis_name="s")
    fn = functools.partial(
        pl.kernel, mesh=mesh,
        out_type=jax.ShapeDtypeStruct((NC, N, D), jnp.float32),
        scratch_types=[
            [pltpu.VMEM((CHUNK,), jnp.int32)] * 3,
            [pltpu.VMEM((CHUNK,), jnp.int32)] * 3,
            [pltpu.VMEM((CHUNK,), jnp.int32)] * 3,
            [pltpu.VMEM((CHUNK,), jnp.int32)] * 3,
            [pltpu.VMEM((CHUNK,), jnp.float32)] * 3,
            [pltpu.VMEM((CHUNK, D), jnp.float32)] * 3,
            pltpu.VMEM((ZROWS, D), jnp.float32),
            pltpu.VMEM_SHARED((N, D), jnp.float32),
            [pltpu.SemaphoreType.DMA] * 3,
            [pltpu.SemaphoreType.DMA] * 3,
            [pltpu.SemaphoreType.DMA] * 3,
        ],
    )(_sc_edge_body)
    return fn(ei_flat, r, norm_flat, h_all)


def _combine_body(p_ref, out_ref):
    out_ref[...] = jnp.maximum(p_ref[0] + p_ref[1], 0.0)


def _combine(partials):
    BLK = 2000
    return pl.pallas_call(
        _combine_body,
        grid=(N // BLK,),
        in_specs=[pl.BlockSpec((NC, BLK, D), lambda bi: (0, bi, 0))],
        out_specs=pl.BlockSpec((BLK, D), lambda bi: (bi, 0)),
        out_shape=jax.ShapeDtypeStruct((N, D), jnp.float32),
    )(partials)


def kernel(edge_index, h, r, norm, W):
    norm_flat = norm.reshape(E)
    h_all = _project(h, W).reshape(R * N, D)
    partials = _sc_edges(edge_index.reshape(2 * E), r, norm_flat, h_all)
    return _combine(partials)


# speedup vs baseline: 2.0914x; 1.0195x over previous
import functools

import jax
import jax.numpy as jnp
from jax import lax
from jax.experimental import pallas as pl
from jax.experimental.pallas import tpu as pltpu
from jax.experimental.pallas import tpu_sc as plsc

N = 10000
D = 128
R = 8
E = 320000

NC = 2
NS = 16
NW = NC * NS
E_PER_W = E // NW
CHUNK = 80
NCHUNK = E_PER_W // CHUNK
NTRIPLE = 40
ZROWS = 16
S_STRIDE = 624
S_ROWS = 640

def _proj_body(h_ref, w_ref, out_ref):
    for rr in range(R):
        out_ref[rr] = jnp.dot(h_ref[...], w_ref[rr],
                              preferred_element_type=jnp.float32)


def _project(h, W):
    BLK = 2000
    return pl.pallas_call(
        _proj_body,
        grid=(N // BLK,),
        in_specs=[
            pl.BlockSpec((BLK, D), lambda bi: (bi, 0)),
            pl.BlockSpec((R, D, D), lambda bi: (0, 0, 0)),
        ],
        out_specs=pl.BlockSpec((R, BLK, D), lambda bi: (0, bi, 0)),
        out_shape=jax.ShapeDtypeStruct((R, N, D), jnp.float32),
    )(h, W)


def _sc_edge_body(ei_hbm, r_hbm, norm_hbm, hall_hbm, out_hbm,
                  srcb, rb, idxc, dstb, nrm, rows,
                  zero_v, agg_sh, gsem, ssem, xsem):
    cid = lax.axis_index("c")
    sid = lax.axis_index("s")
    wid = cid * NS + sid
    base = wid * E_PER_W

    def zero_body(i, carry):
        for c in range(D // 16):
            zero_v[i, pl.ds(c * 16, 16)] = jnp.zeros((16,), jnp.float32)
        return carry
    lax.fori_loop(0, ZROWS, zero_body, 0)

    def zcopy_body(j, carry):
        pltpu.sync_copy(
            zero_v, agg_sh.at[pl.ds(sid * S_STRIDE + j * ZROWS, ZROWS)])
        return carry
    lax.fori_loop(0, S_ROWS // ZROWS, zcopy_body, 0)
    plsc.subcore_barrier()

    def stage_off(t):
        return jnp.minimum(t, NCHUNK - 1) * CHUNK

    def issue_stage(t, b):
        off = stage_off(t)
        pltpu.async_copy(
            ei_hbm.at[pl.ds(base + off, CHUNK)], srcb[b], xsem[b])
        pltpu.async_copy(
            r_hbm.at[pl.ds(base + off, CHUNK)], rb[b], xsem[b])

    def wait_stage(t, b):
        off = stage_off(t)
        pltpu.make_async_copy(
            ei_hbm.at[pl.ds(base + off, CHUNK)], srcb[b], xsem[b]).wait()
        pltpu.make_async_copy(
            r_hbm.at[pl.ds(base + off, CHUNK)], rb[b], xsem[b]).wait()

    def compute_idx(b):
        for g in range(CHUNK // 16):
            s = srcb[b][pl.ds(g * 16, 16)]
            rr = rb[b][pl.ds(g * 16, 16)]
            idxc[b][pl.ds(g * 16, 16)] = rr * N + s

    def issue_g(t, b):
        off = t * CHUNK
        pltpu.async_copy(hall_hbm.at[idxc[b]], rows[b], gsem[b])
        pltpu.async_copy(
            norm_hbm.at[pl.ds(base + off, CHUNK)], nrm[b], gsem[b])
        pltpu.async_copy(
            ei_hbm.at[pl.ds(E + base + off, CHUNK)], dstb[b], gsem[b])

    def wait_g(t, b):
        off = t * CHUNK
        pltpu.make_async_copy(hall_hbm.at[idxc[b]], rows[b],
                              gsem[b]).wait()
        pltpu.make_async_copy(
            norm_hbm.at[pl.ds(base + off, CHUNK)], nrm[b], gsem[b]).wait()
        pltpu.make_async_copy(
            ei_hbm.at[pl.ds(E + base + off, CHUNK)], dstb[b],
            gsem[b]).wait()

    def scale(b):
        def group_body(g, c2):
            nv = nrm[b][pl.ds(g * 16, 16)]
            for k in range(16):
                nb = nv[k]
                e = g * 16 + k
                for c in range(D // 16):
                    rows[b][e, pl.ds(c * 16, 16)] = (
                        rows[b][e, pl.ds(c * 16, 16)] * nb)
            return c2
        lax.fori_loop(0, CHUNK // 16, group_body, 0)

    def issue_s(b):
        pltpu.async_copy(rows[b], agg_sh.at[dstb[b]], ssem[b], add=True)

    def wait_s(b):
        pltpu.make_async_copy(rows[b], agg_sh.at[dstb[b]],
                              ssem[b]).wait()

    for b in range(3):
        issue_stage(b, b)
    for b in range(3):
        wait_stage(b, b)
        compute_idx(b)
        issue_stage(b + 3, b)
        issue_g(b, b)
    wait_g(0, 0); scale(0); issue_s(0)
    wait_stage(3, 0); compute_idx(0); issue_stage(6, 0)

    def station(u, b, b2, last_g, do_stage):
        wait_g(u, b); scale(b); issue_s(b)
        wait_stage(u + 3, b)
        compute_idx(b)
        if do_stage:
            issue_stage(u + 6, b)
        if last_g is not None:
            wait_s(b2)
            issue_g(last_g, b2)

    def triple_body(i, carry):
        u = 3 * i + 1
        station(u, 1, 0, u + 2, True)
        station(u + 1, 2, 1, u + 3, True)
        station(u + 2, 0, 2, u + 4, True)
        return carry
    lax.fori_loop(0, NTRIPLE, triple_body, 0)

    wait_g(121, 1); scale(1); issue_s(1)
    wait_stage(124, 1); compute_idx(1)
    wait_s(0); issue_g(123, 0)
    wait_g(122, 2); scale(2); issue_s(2)
    wait_stage(125, 2)
    wait_s(1); issue_g(124, 1)
    wait_g(123, 0); scale(0); issue_s(0)
    wait_stage(126, 0)
    wait_s(2)
    wait_g(124, 1); scale(1); issue_s(1)
    wait_s(0)
    wait_s(1)

    plsc.subcore_barrier()
    pltpu.sync_copy(
        agg_sh.at[pl.ds(sid * S_STRIDE, S_ROWS)],
        out_hbm.at[cid, pl.ds(sid * S_STRIDE, S_ROWS)])


def _sc_edges(ei_flat, r, norm_flat, h_all):
    mesh = plsc.VectorSubcoreMesh(core_axis_name="c", subcore_axis_name="s")
    fn = functools.partial(
        pl.kernel, mesh=mesh,
        out_type=jax.ShapeDtypeStruct((NC, N, D), jnp.float32),
        scratch_types=[
            [pltpu.VMEM((CHUNK,), jnp.int32)] * 3,
            [pltpu.VMEM((CHUNK,), jnp.int32)] * 3,
            [pltpu.VMEM((CHUNK,), jnp.int32)] * 3,
            [pltpu.VMEM((CHUNK,), jnp.int32)] * 3,
            [pltpu.VMEM((CHUNK,), jnp.float32)] * 3,
            [pltpu.VMEM((CHUNK, D), jnp.float32)] * 3,
            pltpu.VMEM((ZROWS, D), jnp.float32),
            pltpu.VMEM_SHARED((N, D), jnp.float32),
            [pltpu.SemaphoreType.DMA] * 3,
            [pltpu.SemaphoreType.DMA] * 3,
            [pltpu.SemaphoreType.DMA] * 3,
        ],
    )(_sc_edge_body)
    return fn(ei_flat, r, norm_flat, h_all)


def _combine_body(p_ref, out_ref):
    out_ref[...] = jnp.maximum(p_ref[0] + p_ref[1], 0.0)


def _combine(partials):
    BLK = 2000
    return pl.pallas_call(
        _combine_body,
        grid=(N // BLK,),
        in_specs=[pl.BlockSpec((NC, BLK, D), lambda bi: (0, bi, 0))],
        out_specs=pl.BlockSpec((BLK, D), lambda bi: (bi, 0)),
        out_shape=jax.ShapeDtypeStruct((N, D), jnp.float32),
    )(partials)


def kernel(edge_index, h, r, norm, W):
    norm_flat = norm.reshape(E)
    h_all = _project(h, W).reshape(R * N, D)
    partials = _sc_edges(edge_index.reshape(2 * E), r, norm_flat, h_all)
    return _combine(partials)
